# baseline (speedup 1.0000x reference)
import functools

import jax
import jax.numpy as jnp
from jax import lax
from jax.experimental import pallas as pl
from jax.experimental.pallas import tpu as pltpu
from jax.experimental.pallas import tpu_sc as plsc

BATCH = 16384
HIST = 200
DIM = 16
NUM_WORKERS = 32
BW = BATCH // NUM_WORKERS
LANES = 128
SUBL = 8
TPW = BW // LANES
QW = TPW * SUBL * LANES


def _make_gather():
    mesh = plsc.VectorSubcoreMesh(core_axis_name="c", subcore_axis_name="s")

    @functools.partial(
        pl.kernel,
        mesh=mesh,
        out_type=jax.ShapeDtypeStruct((HIST, 2, NUM_WORKERS * QW), jnp.float32),
        scratch_types=[
            pltpu.VMEM((2, BW), jnp.int32),
            pltpu.VMEM((2, BW, DIM), jnp.float32),
            pltpu.VMEM((2, 2 * QW), jnp.float32),
            pltpu.SemaphoreType.DMA,
            pltpu.SemaphoreType.DMA,
            pltpu.SemaphoreType.DMA,
            pltpu.SemaphoreType.DMA,
            pltpu.SemaphoreType.DMA,
            pltpu.SemaphoreType.DMA,
        ],
        compiler_params=pltpu.CompilerParams(use_tc_tiling_on_sc=False, needs_layout_passes=False),
    )
    def gather_kernel(idx_hbm, table_hbm, out_hbm, idx_v, rows_v, tiles_v,
                      si0, si1, sg0, sg1, ss0, ss1):
        wid = lax.axis_index("s") * 2 + lax.axis_index("c")
        si = (si0, si1)
        sg = (sg0, sg1)
        ss = (ss0, ss1)
        N = HIST


        def idx_copy(h, b):
            return pltpu.make_async_copy(
                idx_hbm.at[h, pl.ds(wid * BW, BW)], idx_v.at[b], si[b])

        def gat_copy(h, b):
            return pltpu.make_async_copy(
                table_hbm.at[idx_v.at[b]], rows_v.at[b], sg[b])

        def st_copies(h, b):
            return [pltpu.make_async_copy(
                        tiles_v.at[b, pl.ds(td * QW, QW)],
                        out_hbm.at[h, td, pl.ds(wid * QW, QW)], ss[b])
                    for td in range(2)]

        d_iota = lax.iota(jnp.int32, DIM)
        pat0 = (d_iota // SUBL) * QW + (d_iota % SUBL) * LANES

        def transpose(b):
            @plsc.parallel_loop(0, BW, unroll=8, carry=pat0)
            def tr_body(r, pat):
                v = rows_v[b, r]
                offs = pat + ((r // LANES) * (SUBL * LANES) + r % LANES)
                plsc.store_scatter(tiles_v.at[b], [offs], v)
                return pat

        idx_copy(0, 0).start()
        idx_copy(1, 1).start()
        idx_copy(0, 0).wait()
        gat_copy(0, 0).start()

        def body(g, carry):
            for b in (0, 1):
                h = 2 * g + b
                other = 1 - b

                @pl.when(h + 1 < N)
                def _():
                    idx_copy(h + 1, other).wait()
                    gat_copy(h + 1, other).start()

                gat_copy(h, b).wait()

                @pl.when(h + 2 < N)
                def _():
                    idx_copy(h + 2, b).start()

                @pl.when(h >= 2)
                def _():
                    for cp in st_copies(h - 2, b):
                        cp.wait()

                transpose(b)
                for cp in st_copies(h, b):
                    cp.start()

            return carry

        lax.fori_loop(0, N // 2, body, 0)
        for cp in st_copies(N - 2, 0):
            cp.wait()
        for cp in st_copies(N - 1, 1):
            cp.wait()

    return gather_kernel


_gather = _make_gather()


def kernel(indices, table):
    idx_t = jnp.swapaxes(indices, 0, 1).astype(jnp.int32)
    out5 = _gather(idx_t, table)
    return (out5.reshape(HIST, 2, BATCH // LANES, SUBL, LANES)
                .transpose(2, 4, 0, 1, 3)
                .reshape(BATCH, HIST, DIM))


# speedup vs baseline: 1.6529x; 1.6529x over previous
import functools

import jax
import jax.numpy as jnp
from jax import lax
from jax.experimental import pallas as pl
from jax.experimental.pallas import tpu as pltpu
from jax.experimental.pallas import tpu_sc as plsc

BATCH = 16384
HIST = 200
DIM = 16
NUM_WORKERS = 32
BW = BATCH // NUM_WORKERS
LANES = 128
SUBL = 8
TPW = BW // LANES
QW = TPW * SUBL * LANES
PADW = LANES + 1


def _make_gather():
    mesh = plsc.VectorSubcoreMesh(core_axis_name="c", subcore_axis_name="s")

    @functools.partial(
        pl.kernel,
        mesh=mesh,
        out_type=jax.ShapeDtypeStruct((HIST, 2, BATCH // LANES * SUBL, LANES),
                                      jnp.float32),
        scratch_types=[
            pltpu.VMEM((2, BW), jnp.int32),
            pltpu.VMEM((2, BW, DIM), jnp.float32),
            pltpu.VMEM((2, 2 * TPW * SUBL, PADW), jnp.float32),
            pltpu.SemaphoreType.DMA,
            pltpu.SemaphoreType.DMA,
            pltpu.SemaphoreType.DMA,
            pltpu.SemaphoreType.DMA,
            pltpu.SemaphoreType.DMA,
            pltpu.SemaphoreType.DMA,
        ],
        compiler_params=pltpu.CompilerParams(use_tc_tiling_on_sc=False, needs_layout_passes=False),
    )
    def gather_kernel(idx_hbm, table_hbm, out_hbm, idx_v, rows_v, tiles_v,
                      si0, si1, sg0, sg1, ss0, ss1):
        wid = lax.axis_index("s") * 2 + lax.axis_index("c")
        si = (si0, si1)
        sg = (sg0, sg1)
        ss = (ss0, ss1)
        N = HIST


        def idx_copy(h, b):
            return pltpu.make_async_copy(
                idx_hbm.at[h, pl.ds(wid * BW, BW)], idx_v.at[b], si[b])

        def gat_copy(h, b):
            return pltpu.make_async_copy(
                table_hbm.at[idx_v.at[b]], rows_v.at[b], sg[b])

        rows_per_td = TPW * SUBL

        def st_copies(h, b):
            return [pltpu.make_async_copy(
                        tiles_v.at[b, pl.ds(td * rows_per_td, rows_per_td),
                                   pl.ds(0, LANES)],
                        out_hbm.at[h, td, pl.ds(wid * rows_per_td, rows_per_td)],
                        ss[b])
                    for td in range(2)]

        d_iota = lax.iota(jnp.int32, DIM)
        pat0 = (d_iota // SUBL) * (TPW * SUBL) + d_iota % SUBL

        def transpose(b):
            @plsc.parallel_loop(0, BW, unroll=8, carry=pat0)
            def tr_body(r, pat):
                v = rows_v[b, r]
                i_row = pat + (r // LANES) * SUBL
                i_col = jnp.zeros((DIM,), jnp.int32) + r % LANES
                plsc.store_scatter(tiles_v.at[b], [i_row, i_col], v)
                return pat

        idx_copy(0, 0).start()
        idx_copy(1, 1).start()
        idx_copy(0, 0).wait()
        gat_copy(0, 0).start()

        def body(g, carry):
            for b in (0, 1):
                h = 2 * g + b
                other = 1 - b

                @pl.when(h + 1 < N)
                def _():
                    idx_copy(h + 1, other).wait()
                    gat_copy(h + 1, other).start()

                gat_copy(h, b).wait()

                @pl.when(h + 2 < N)
                def _():
                    idx_copy(h + 2, b).start()

                @pl.when(h >= 2)
                def _():
                    for cp in st_copies(h - 2, b):
                        cp.wait()

                transpose(b)
                for cp in st_copies(h, b):
                    cp.start()

            return carry

        lax.fori_loop(0, N // 2, body, 0)
        for cp in st_copies(N - 2, 0):
            cp.wait()
        for cp in st_copies(N - 1, 1):
            cp.wait()

    return gather_kernel


_gather = _make_gather()


def kernel(indices, table):
    idx_t = jnp.swapaxes(indices, 0, 1).astype(jnp.int32)
    out5 = _gather(idx_t, table)
    return (out5.reshape(HIST, 2, BATCH // LANES, SUBL, LANES)
                .transpose(2, 4, 0, 1, 3)
                .reshape(BATCH, HIST, DIM))


# speedup vs baseline: 1.6601x; 1.0044x over previous
import functools

import jax
import jax.numpy as jnp
from jax import lax
from jax.experimental import pallas as pl
from jax.experimental.pallas import tpu as pltpu
from jax.experimental.pallas import tpu_sc as plsc

BATCH = 16384
HIST = 200
DIM = 16
NUM_WORKERS = 32
BW = BATCH // NUM_WORKERS
LANES = 128
SUBL = 8
TPW = BW // LANES
QW = TPW * SUBL * LANES
PADW = LANES + 1


def _make_gather():
    mesh = plsc.VectorSubcoreMesh(core_axis_name="c", subcore_axis_name="s")

    @functools.partial(
        pl.kernel,
        mesh=mesh,
        out_type=jax.ShapeDtypeStruct((HIST, 2, BATCH // LANES * SUBL, LANES),
                                      jnp.float32),
        scratch_types=[
            pltpu.VMEM((2, BW), jnp.int32),
            pltpu.VMEM((2, BW, DIM), jnp.float32),
            pltpu.VMEM((2, 2 * TPW * SUBL, PADW), jnp.float32),
            pltpu.SemaphoreType.DMA,
            pltpu.SemaphoreType.DMA,
            pltpu.SemaphoreType.DMA,
            pltpu.SemaphoreType.DMA,
            pltpu.SemaphoreType.DMA,
            pltpu.SemaphoreType.DMA,
        ],
        compiler_params=pltpu.CompilerParams(use_tc_tiling_on_sc=False, needs_layout_passes=False),
    )
    def gather_kernel(idx_hbm, table_hbm, out_hbm, idx_v, rows_v, tiles_v,
                      si0, si1, sg0, sg1, ss0, ss1):
        wid = lax.axis_index("s") * 2 + lax.axis_index("c")
        si = (si0, si1)
        sg = (sg0, sg1)
        ss = (ss0, ss1)
        N = HIST


        def idx_copies(h, b):
            return [pltpu.make_async_copy(
                        idx_hbm.at[h // SUBL, TPW * wid + j, h % SUBL],
                        idx_v.at[b, pl.ds(j * LANES, LANES)], si[b])
                    for j in range(TPW)]

        def idx_start(h, b):
            for cp in idx_copies(h, b):
                cp.start()

        def idx_wait(h, b):
            for cp in idx_copies(h, b):
                cp.wait()

        def gat_copy(h, b):
            return pltpu.make_async_copy(
                table_hbm.at[idx_v.at[b]], rows_v.at[b], sg[b])

        rows_per_td = TPW * SUBL

        def st_copies(h, b):
            return [pltpu.make_async_copy(
                        tiles_v.at[b, pl.ds(td * rows_per_td, rows_per_td),
                                   pl.ds(0, LANES)],
                        out_hbm.at[h, td, pl.ds(wid * rows_per_td, rows_per_td)],
                        ss[b])
                    for td in range(2)]

        d_iota = lax.iota(jnp.int32, DIM)
        pat0 = (d_iota // SUBL) * (TPW * SUBL) + d_iota % SUBL

        def transpose(b):
            @plsc.parallel_loop(0, BW, unroll=8, carry=pat0)
            def tr_body(r, pat):
                v = rows_v[b, r]
                i_row = pat + (r // LANES) * SUBL
                i_col = jnp.zeros((DIM,), jnp.int32) + r % LANES
                plsc.store_scatter(tiles_v.at[b], [i_row, i_col], v)
                return pat

        idx_start(0, 0)
        idx_start(1, 1)
        idx_wait(0, 0)
        gat_copy(0, 0).start()

        def body(g, carry):
            for b in (0, 1):
                h = 2 * g + b
                other = 1 - b

                @pl.when(h + 1 < N)
                def _():
                    idx_wait(h + 1, other)
                    gat_copy(h + 1, other).start()

                gat_copy(h, b).wait()

                @pl.when(h + 2 < N)
                def _():
                    idx_start(h + 2, b)

                @pl.when(h >= 2)
                def _():
                    for cp in st_copies(h - 2, b):
                        cp.wait()

                transpose(b)
                for cp in st_copies(h, b):
                    cp.start()

            return carry

        lax.fori_loop(0, N // 2, body, 0)
        for cp in st_copies(N - 2, 0):
            cp.wait()
        for cp in st_copies(N - 1, 1):
            cp.wait()

    return gather_kernel


_gather = _make_gather()


def kernel(indices, table):
    idx4 = (indices.astype(jnp.int32)
            .reshape(LANES, LANES, HIST // SUBL, SUBL)
            .transpose(2, 0, 3, 1))
    out5 = _gather(idx4, table)
    return (out5.reshape(HIST, 2, BATCH // LANES, SUBL, LANES)
                .transpose(2, 4, 0, 1, 3)
                .reshape(BATCH, HIST, DIM))


# speedup vs baseline: 1.7401x; 1.0482x over previous
import functools

import jax
import jax.numpy as jnp
from jax import lax
from jax.experimental import pallas as pl
from jax.experimental.pallas import tpu as pltpu
from jax.experimental.pallas import tpu_sc as plsc

BATCH = 16384
HIST = 200
DIM = 16
VOCAB = 1000000
NUM_WORKERS = 32
BW = BATCH // NUM_WORKERS
LANES = 128
SUBL = 8
TPW = BW // LANES
QW = TPW * SUBL * LANES
PADW = LANES + 1

VPAD = 1 << 20
TC4 = VPAD // LANES
NTC = -(-VOCAB // LANES)
TCH = 8
ROWS1 = TCH * LANES
NCH1 = -(-NTC // TCH)
K1 = -(-NCH1 // 16)
PADT = DIM + 1


def _make_gather():
    mesh = plsc.VectorSubcoreMesh(core_axis_name="c", subcore_axis_name="s")

    @functools.partial(
        pl.kernel,
        mesh=mesh,
        out_type=[
            jax.ShapeDtypeStruct((HIST, 2, BATCH // LANES * SUBL, LANES),
                                 jnp.float32),
            jax.ShapeDtypeStruct((2 * VPAD, DIM), jnp.float32),
        ],
        scratch_types=[
            pltpu.VMEM((2, BW), jnp.int32),
            pltpu.VMEM((2, BW, DIM), jnp.float32),
            pltpu.VMEM((2, 2 * TPW * SUBL, PADW), jnp.float32),
            pltpu.VMEM((2, 2, TCH, SUBL, LANES), jnp.float32),
            pltpu.VMEM((2, ROWS1, PADT), jnp.float32),
            pltpu.SemaphoreType.DMA,
            pltpu.SemaphoreType.DMA,
            pltpu.SemaphoreType.DMA,
            pltpu.SemaphoreType.DMA,
            pltpu.SemaphoreType.DMA,
            pltpu.SemaphoreType.DMA,
        ],
        compiler_params=pltpu.CompilerParams(use_tc_tiling_on_sc=False,
                                             needs_layout_passes=False),
    )
    def gather_kernel(idx_hbm, tab4_hbm, out_hbm, tbl_hbm,
                      idx_v, rows_v, tiles_v, slab_v, stage_v,
                      si0, si1, sg0, sg1, ss0, ss1):
        cid = lax.axis_index("c")
        sid = lax.axis_index("s")
        wid = sid * 2 + cid
        tbase = cid * VPAD
        si = (si0, si1)
        sg = (sg0, sg1)
        ss = (ss0, ss1)
        d_iota = lax.iota(jnp.int32, DIM)

        def p1_in(q, b):
            return pltpu.make_async_copy(
                tab4_hbm.at[:, pl.ds(q * TCH, TCH)], slab_v.at[b], si[b])

        def p1_out(q, b):
            return pltpu.make_async_copy(
                stage_v.at[b, pl.ds(0, ROWS1), pl.ds(0, DIM)],
                tbl_hbm.at[pl.ds(tbase + q * ROWS1, ROWS1)], ss[b])

        def p1_transpose(b):
            @plsc.parallel_loop(0, TCH * SUBL * DIM, unroll=8, carry=d_iota)
            def p1_body(t, iot):
                d = t % DIM
                g = t // DIM
                tcl = g // SUBL
                lg = g % SUBL
                v = slab_v[b, d // SUBL, tcl, d % SUBL, pl.ds(lg * 16, 16)]
                i_row = iot + (tcl * LANES + lg * 16)
                i_col = iot * 0 + d
                plsc.store_scatter(stage_v.at[b], [i_row, i_col], v)
                return iot

        def p1_q(k):
            return sid + 16 * k

        p1_in(p1_q(0), 0).start()

        def p1_body_k(k, carry):
            for b in (0, 1):
                kk = 2 * k + b
                q = p1_q(kk)

                @pl.when(q < NCH1)
                def _():
                    p1_in(q, b).wait()

                    @pl.when(p1_q(kk + 1) < NCH1)
                    def _():
                        p1_in(p1_q(kk + 1), 1 - b).start()

                    @pl.when(kk >= 2)
                    def _():
                        p1_out(p1_q(kk - 2), b).wait()

                    p1_transpose(b)
                    p1_out(q, b).start()

            return carry

        lax.fori_loop(0, (K1 + 1) // 2, p1_body_k, 0)
        for tail in (K1 - 2, K1 - 1):
            @pl.when(p1_q(tail) < NCH1)
            def _(tail=tail):
                p1_out(p1_q(tail), tail % 2).wait()

        plsc.subcore_barrier()

        N = HIST
        pat0 = (d_iota // SUBL) * (TPW * SUBL) + d_iota % SUBL
        rows_per_td = TPW * SUBL

        def idx_copies(h, b):
            return [pltpu.make_async_copy(
                        idx_hbm.at[h // SUBL, TPW * wid + j, h % SUBL],
                        idx_v.at[b, pl.ds(j * LANES, LANES)], si[b])
                    for j in range(TPW)]

        def idx_start(h, b):
            for cp in idx_copies(h, b):
                cp.start()

        def idx_wait(h, b):
            for cp in idx_copies(h, b):
                cp.wait()

        def idx_adjust(b):
            @plsc.parallel_loop(0, BW // DIM, unroll=8)
            def adj_body(i):
                sl = pl.ds(i * DIM, DIM)
                idx_v[b, sl] = idx_v[b, sl] + tbase

        def gat_copy(h, b):
            return pltpu.make_async_copy(
                tbl_hbm.at[idx_v.at[b]], rows_v.at[b], sg[b])

        def st_copies(h, b):
            return [pltpu.make_async_copy(
                        tiles_v.at[b, pl.ds(td * rows_per_td, rows_per_td),
                                   pl.ds(0, LANES)],
                        out_hbm.at[h, td, pl.ds(wid * rows_per_td,
                                                rows_per_td)],
                        ss[b])
                    for td in range(2)]

        def transpose(b):
            @plsc.parallel_loop(0, BW, unroll=8, carry=pat0)
            def tr_body(r, pat):
                v = rows_v[b, r]
                i_row = pat + (r // LANES) * SUBL
                i_col = jnp.zeros((DIM,), jnp.int32) + r % LANES
                plsc.store_scatter(tiles_v.at[b], [i_row, i_col], v)
                return pat

        idx_start(0, 0)
        idx_start(1, 1)
        idx_wait(0, 0)
        idx_adjust(0)
        gat_copy(0, 0).start()

        def body(g, carry):
            for b in (0, 1):
                h = 2 * g + b
                other = 1 - b

                @pl.when(h + 1 < N)
                def _():
                    idx_wait(h + 1, other)
                    idx_adjust(other)
                    gat_copy(h + 1, other).start()

                gat_copy(h, b).wait()

                @pl.when(h + 2 < N)
                def _():
                    idx_start(h + 2, b)

                @pl.when(h >= 2)
                def _():
                    for cp in st_copies(h - 2, b):
                        cp.wait()

                transpose(b)
                for cp in st_copies(h, b):
                    cp.start()

            return carry

        lax.fori_loop(0, N // 2, body, 0)
        for cp in st_copies(N - 2, 0):
            cp.wait()
        for cp in st_copies(N - 1, 1):
            cp.wait()

    return gather_kernel


_gather = _make_gather()


def kernel(indices, table):
    idx4 = (indices.astype(jnp.int32)
            .reshape(LANES, LANES, HIST // SUBL, SUBL)
            .transpose(2, 0, 3, 1))
    tpad = jnp.pad(table, ((0, VPAD - VOCAB), (0, 0)))
    tab4 = tpad.reshape(TC4, LANES, 2, SUBL).transpose(2, 0, 3, 1)
    out4, _ = _gather(idx4, tab4)
    return (out4.reshape(HIST, 2, BATCH // LANES, SUBL, LANES)
                .transpose(2, 4, 0, 1, 3)
                .reshape(BATCH, HIST, DIM))


# speedup vs baseline: 2.6910x; 1.5465x over previous
import functools

import jax
import jax.numpy as jnp
from jax import lax
from jax.experimental import pallas as pl
from jax.experimental.pallas import tpu as pltpu
from jax.experimental.pallas import tpu_sc as plsc

BATCH = 16384
HIST = 200
DIM = 16
VOCAB = 1000000
NUM_WORKERS = 32
BW = BATCH // NUM_WORKERS
LANES = 128
SUBL = 8
TPW = BW // LANES
QW = TPW * SUBL * LANES
PADW = LANES + 1

VPAD = 1 << 20
TC4 = VPAD // LANES
NTC = -(-VOCAB // LANES)
TCH = 8
ROWS1 = TCH * LANES
NCH1 = -(-NTC // TCH)
K1 = -(-NCH1 // 16)
PADT = DIM + 1


def _make_gather():
    mesh = plsc.VectorSubcoreMesh(core_axis_name="c", subcore_axis_name="s")

    @functools.partial(
        pl.kernel,
        mesh=mesh,
        out_type=[
            jax.ShapeDtypeStruct((HIST, 2, BATCH // LANES * SUBL, LANES),
                                 jnp.float32),
            jax.ShapeDtypeStruct((2 * VPAD, PADT), jnp.float32),
        ],
        scratch_types=[
            pltpu.VMEM((2, BW), jnp.int32),
            pltpu.VMEM((2, BW, PADT), jnp.float32),
            pltpu.VMEM((2, 2 * TPW * SUBL, PADW), jnp.float32),
            pltpu.VMEM((2, 2, TCH, SUBL, LANES), jnp.float32),
            pltpu.VMEM((2, ROWS1, PADT), jnp.float32),
            pltpu.SemaphoreType.DMA,
            pltpu.SemaphoreType.DMA,
            pltpu.SemaphoreType.DMA,
            pltpu.SemaphoreType.DMA,
            pltpu.SemaphoreType.DMA,
            pltpu.SemaphoreType.DMA,
        ],
        compiler_params=pltpu.CompilerParams(use_tc_tiling_on_sc=False,
                                             needs_layout_passes=False),
    )
    def gather_kernel(idx_hbm, tab4_hbm, out_hbm, tbl_hbm,
                      idx_v, rows_v, tiles_v, slab_v, stage_v,
                      si0, si1, sg0, sg1, ss0, ss1):
        cid = lax.axis_index("c")
        sid = lax.axis_index("s")
        wid = sid * 2 + cid
        tbase = cid * VPAD
        si = (si0, si1)
        sg = (sg0, sg1)
        ss = (ss0, ss1)
        d_iota = lax.iota(jnp.int32, DIM)

        def p1_in(q, b):
            return pltpu.make_async_copy(
                tab4_hbm.at[:, pl.ds(q * TCH, TCH)], slab_v.at[b], si[b])

        def p1_out(q, b):
            return pltpu.make_async_copy(
                stage_v.at[b], tbl_hbm.at[pl.ds(tbase + q * ROWS1, ROWS1)],
                ss[b])

        def p1_transpose(b):
            @plsc.parallel_loop(0, TCH * SUBL * DIM, unroll=8, carry=d_iota)
            def p1_body(t, iot):
                d = t % DIM
                g = t // DIM
                tcl = g // SUBL
                lg = g % SUBL
                v = slab_v[b, d // SUBL, tcl, d % SUBL, pl.ds(lg * 16, 16)]
                i_row = iot + (tcl * LANES + lg * 16)
                i_col = iot * 0 + d
                plsc.store_scatter(stage_v.at[b], [i_row, i_col], v)
                return iot

        def p1_q(k):
            return sid + 16 * k

        p1_in(p1_q(0), 0).start()

        def p1_body_k(k, carry):
            for b in (0, 1):
                kk = 2 * k + b
                q = p1_q(kk)

                @pl.when(q < NCH1)
                def _():
                    p1_in(q, b).wait()

                    @pl.when(p1_q(kk + 1) < NCH1)
                    def _():
                        p1_in(p1_q(kk + 1), 1 - b).start()

                    @pl.when(kk >= 2)
                    def _():
                        p1_out(p1_q(kk - 2), b).wait()

                    p1_transpose(b)
                    p1_out(q, b).start()

            return carry

        lax.fori_loop(0, (K1 + 1) // 2, p1_body_k, 0)
        for tail in (K1 - 2, K1 - 1):
            @pl.when(p1_q(tail) < NCH1)
            def _(tail=tail):
                p1_out(p1_q(tail), tail % 2).wait()

        plsc.subcore_barrier()

        N = HIST
        pat0 = (d_iota // SUBL) * (TPW * SUBL) + d_iota % SUBL
        rows_per_td = TPW * SUBL

        def idx_copies(h, b):
            return [pltpu.make_async_copy(
                        idx_hbm.at[h // SUBL, TPW * wid + j, h % SUBL],
                        idx_v.at[b, pl.ds(j * LANES, LANES)], si[b])
                    for j in range(TPW)]

        def idx_start(h, b):
            for cp in idx_copies(h, b):
                cp.start()

        def idx_wait(h, b):
            for cp in idx_copies(h, b):
                cp.wait()

        def idx_adjust(b):
            @plsc.parallel_loop(0, BW // DIM, unroll=8)
            def adj_body(i):
                sl = pl.ds(i * DIM, DIM)
                idx_v[b, sl] = idx_v[b, sl] + tbase

        def gat_copy(h, b):
            return pltpu.make_async_copy(
                tbl_hbm.at[idx_v.at[b]], rows_v.at[b], sg[b])

        def st_copies(h, b):
            return [pltpu.make_async_copy(
                        tiles_v.at[b, pl.ds(td * rows_per_td, rows_per_td),
                                   pl.ds(0, LANES)],
                        out_hbm.at[h, td, pl.ds(wid * rows_per_td,
                                                rows_per_td)],
                        ss[b])
                    for td in range(2)]

        def transpose(b):
            @plsc.parallel_loop(0, BW, unroll=8, carry=pat0)
            def tr_body(r, pat):
                v = rows_v[b, r, pl.ds(0, DIM)]
                i_row = pat + (r // LANES) * SUBL
                i_col = jnp.zeros((DIM,), jnp.int32) + r % LANES
                plsc.store_scatter(tiles_v.at[b], [i_row, i_col], v)
                return pat

        idx_start(0, 0)
        idx_start(1, 1)
        idx_wait(0, 0)
        idx_adjust(0)
        gat_copy(0, 0).start()

        def body(g, carry):
            for b in (0, 1):
                h = 2 * g + b
                other = 1 - b

                @pl.when(h + 1 < N)
                def _():
                    idx_wait(h + 1, other)
                    idx_adjust(other)
                    gat_copy(h + 1, other).start()

                gat_copy(h, b).wait()

                @pl.when(h + 2 < N)
                def _():
                    idx_start(h + 2, b)

                @pl.when(h >= 2)
                def _():
                    for cp in st_copies(h - 2, b):
                        cp.wait()

                transpose(b)
                for cp in st_copies(h, b):
                    cp.start()

            return carry

        lax.fori_loop(0, N // 2, body, 0)
        for cp in st_copies(N - 2, 0):
            cp.wait()
        for cp in st_copies(N - 1, 1):
            cp.wait()

    return gather_kernel


_gather = _make_gather()


def kernel(indices, table):
    idx4 = (indices.astype(jnp.int32)
            .reshape(LANES, LANES, HIST // SUBL, SUBL)
            .transpose(2, 0, 3, 1))
    tpad = jnp.pad(table, ((0, VPAD - VOCAB), (0, 0)))
    tab4 = tpad.reshape(TC4, LANES, 2, SUBL).transpose(2, 0, 3, 1)
    out4, _ = _gather(idx4, tab4)
    return (out4.reshape(HIST, 2, BATCH // LANES, SUBL, LANES)
                .transpose(2, 4, 0, 1, 3)
                .reshape(BATCH, HIST, DIM))
